# initial kernel scaffold (unmeasured)
import jax
import jax.numpy as jnp
from jax import lax
from jax.experimental import pallas as pl
from jax.experimental.pallas import tpu as pltpu

N_DEV = 4


def _layer_body(x_ref, win_ref, wout_ref, out_ref,
                xfull_ref, acc_ref, comm_ref,
                ag_send, ag_recv, rs_send, rs_recv):
    m_per, d = x_ref.shape
    my = lax.axis_index("i")
    left = lax.rem(my + N_DEV - 1, N_DEV)
    right = lax.rem(my + 1, N_DEV)

    barrier = pltpu.get_barrier_semaphore()
    for nbr in (left, right):
        pl.semaphore_signal(barrier, inc=1, device_id=(nbr,),
                            device_id_type=pl.DeviceIdType.MESH)
    pl.semaphore_wait(barrier, 2)

    xfull_ref[pl.ds(my * m_per, m_per), :] = x_ref[:, :]

    for h in range(N_DEV - 1):
        c = lax.rem(my + N_DEV - h, N_DEV)
        rdma = pltpu.make_async_remote_copy(
            src_ref=xfull_ref.at[pl.ds(c * m_per, m_per), :],
            dst_ref=xfull_ref.at[pl.ds(c * m_per, m_per), :],
            send_sem=ag_send.at[h],
            recv_sem=ag_recv.at[h],
            device_id=(right,),
            device_id_type=pl.DeviceIdType.MESH,
        )
        rdma.start()
        rdma.wait()

    hdn = jnp.maximum(
        jnp.dot(xfull_ref[:, :], win_ref[:, :],
                preferred_element_type=jnp.float32),
        0.0,
    )
    acc_ref[:, :] = jnp.dot(hdn, wout_ref[:, :],
                            preferred_element_type=jnp.float32)

    for t in range(N_DEV - 1):
        cs = lax.rem(my + 2 * N_DEV - 1 - t, N_DEV)
        rdma = pltpu.make_async_remote_copy(
            src_ref=acc_ref.at[pl.ds(cs * m_per, m_per), :],
            dst_ref=comm_ref.at[t],
            send_sem=rs_send.at[t],
            recv_sem=rs_recv.at[t],
            device_id=(right,),
            device_id_type=pl.DeviceIdType.MESH,
        )
        rdma.start()
        rdma.wait()
        cr = lax.rem(my + 2 * N_DEV - 2 - t, N_DEV)
        acc_ref[pl.ds(cr * m_per, m_per), :] = (
            acc_ref[pl.ds(cr * m_per, m_per), :] + comm_ref[t])

    out_ref[:, :] = acc_ref[pl.ds(my * m_per, m_per), :]


def _layer(x_shard, win, wout, cid):
    m_per, d = x_shard.shape
    return pl.pallas_call(
        _layer_body,
        out_shape=jax.ShapeDtypeStruct((m_per, d), jnp.float32),
        in_specs=[pl.BlockSpec(memory_space=pltpu.VMEM)] * 3,
        out_specs=pl.BlockSpec(memory_space=pltpu.VMEM),
        scratch_shapes=[
            pltpu.VMEM((N_DEV * m_per, d), jnp.float32),
            pltpu.VMEM((N_DEV * m_per, d), jnp.float32),
            pltpu.VMEM((N_DEV - 1, m_per, d), jnp.float32),
            pltpu.SemaphoreType.DMA((N_DEV - 1,)),
            pltpu.SemaphoreType.DMA((N_DEV - 1,)),
            pltpu.SemaphoreType.DMA((N_DEV - 1,)),
            pltpu.SemaphoreType.DMA((N_DEV - 1,)),
        ],
        compiler_params=pltpu.CompilerParams(collective_id=cid),
    )(x_shard, win, wout)


def kernel(x, Win0, Wout0, Win1, Wout1, Win2, Wout2):
    x = _layer(x, Win0, Wout0, 0)
    x = _layer(x, Win1, Wout1, 1)
    x = _layer(x, Win2, Wout2, 2)
    return x


# baseline (device time: 211490 ns/iter reference)
import jax
import jax.numpy as jnp
from jax import lax
from jax.experimental import pallas as pl
from jax.experimental.pallas import tpu as pltpu

N_DEV = 4
K_TILES = 8


def _layer_body(x_ref, win_ref, wout_ref, out_ref,
                xfull_ref, acc_ref, comm_ref,
                ag_send, ag_recv, rs_send, rs_recv):
    k = pl.program_id(0)
    m_per, d = x_ref.shape
    my = lax.axis_index("i")
    left = lax.rem(my + N_DEV - 1, N_DEV)
    right = lax.rem(my + 1, N_DEV)

    @pl.when(k == 0)
    def _ag():
        barrier = pltpu.get_barrier_semaphore()
        for nbr in (left, right):
            pl.semaphore_signal(barrier, inc=1, device_id=(nbr,),
                                device_id_type=pl.DeviceIdType.MESH)
        pl.semaphore_wait(barrier, 2)

        xfull_ref[pl.ds(my * m_per, m_per), :] = x_ref[:, :]

        for h in range(N_DEV - 1):
            c = lax.rem(my + N_DEV - h, N_DEV)
            rdma = pltpu.make_async_remote_copy(
                src_ref=xfull_ref.at[pl.ds(c * m_per, m_per), :],
                dst_ref=xfull_ref.at[pl.ds(c * m_per, m_per), :],
                send_sem=ag_send.at[h],
                recv_sem=ag_recv.at[h],
                device_id=(right,),
                device_id_type=pl.DeviceIdType.MESH,
            )
            rdma.start()
            rdma.wait()

    hdn = jnp.maximum(
        jnp.dot(xfull_ref[:, :], win_ref[:, :],
                preferred_element_type=jnp.float32),
        0.0,
    )
    contrib = jnp.dot(hdn, wout_ref[:, :], preferred_element_type=jnp.float32)

    @pl.when(k == 0)
    def _init():
        acc_ref[:, :] = contrib

    @pl.when(k > 0)
    def _accum():
        acc_ref[:, :] = acc_ref[:, :] + contrib

    @pl.when(k == K_TILES - 1)
    def _rs():
        for t in range(N_DEV - 1):
            cs = lax.rem(my + 2 * N_DEV - 1 - t, N_DEV)
            rdma = pltpu.make_async_remote_copy(
                src_ref=acc_ref.at[pl.ds(cs * m_per, m_per), :],
                dst_ref=comm_ref.at[t],
                send_sem=rs_send.at[t],
                recv_sem=rs_recv.at[t],
                device_id=(right,),
                device_id_type=pl.DeviceIdType.MESH,
            )
            rdma.start()
            rdma.wait()
            cr = lax.rem(my + 2 * N_DEV - 2 - t, N_DEV)
            acc_ref[pl.ds(cr * m_per, m_per), :] = (
                acc_ref[pl.ds(cr * m_per, m_per), :] + comm_ref[t])

        out_ref[:, :] = acc_ref[pl.ds(my * m_per, m_per), :]


def _layer(x_shard, win, wout, cid):
    m_per, d = x_shard.shape
    h_loc = win.shape[1]
    h_tile = h_loc // K_TILES
    return pl.pallas_call(
        _layer_body,
        grid=(K_TILES,),
        out_shape=jax.ShapeDtypeStruct((m_per, d), jnp.float32),
        in_specs=[
            pl.BlockSpec((m_per, d), lambda k: (0, 0)),
            pl.BlockSpec((d, h_tile), lambda k: (0, k)),
            pl.BlockSpec((h_tile, d), lambda k: (k, 0)),
        ],
        out_specs=pl.BlockSpec((m_per, d), lambda k: (0, 0)),
        scratch_shapes=[
            pltpu.VMEM((N_DEV * m_per, d), jnp.float32),
            pltpu.VMEM((N_DEV * m_per, d), jnp.float32),
            pltpu.VMEM((N_DEV - 1, m_per, d), jnp.float32),
            pltpu.SemaphoreType.DMA((N_DEV - 1,)),
            pltpu.SemaphoreType.DMA((N_DEV - 1,)),
            pltpu.SemaphoreType.DMA((N_DEV - 1,)),
            pltpu.SemaphoreType.DMA((N_DEV - 1,)),
        ],
        compiler_params=pltpu.CompilerParams(
            collective_id=cid,
            dimension_semantics=("arbitrary",),
        ),
    )(x_shard, win, wout)


def kernel(x, Win0, Wout0, Win1, Wout1, Win2, Wout2):
    x = _layer(x, Win0, Wout0, 0)
    x = _layer(x, Win1, Wout1, 1)
    x = _layer(x, Win2, Wout2, 2)
    return x


# device time: 160426 ns/iter; 1.3183x vs baseline; 1.3183x over previous
import jax
import jax.numpy as jnp
from jax import lax
from jax.experimental import pallas as pl
from jax.experimental.pallas import tpu as pltpu

N_DEV = 4
K_TILES = 8


def _layer_body(x_ref, win_ref, wout_ref, out_ref,
                xfull_ref, acc_ref, comm_ref,
                ag_send, ag_recv, rs_send, rs_recv):
    k = pl.program_id(0)
    m_per, d = x_ref.shape
    my = lax.axis_index("i")

    @pl.when(k == 0)
    def _ag():
        xfull_ref[pl.ds(my * m_per, m_per), :] = x_ref[:, :]

        barrier = pltpu.get_barrier_semaphore()
        for off in range(1, N_DEV):
            peer = lax.rem(my + off, N_DEV)
            pl.semaphore_signal(barrier, inc=1, device_id=(peer,),
                                device_id_type=pl.DeviceIdType.MESH)
        pl.semaphore_wait(barrier, N_DEV - 1)

        rdmas = []
        for off in range(1, N_DEV):
            peer = lax.rem(my + off, N_DEV)
            rdma = pltpu.make_async_remote_copy(
                src_ref=xfull_ref.at[pl.ds(my * m_per, m_per), :],
                dst_ref=xfull_ref.at[pl.ds(my * m_per, m_per), :],
                send_sem=ag_send.at[off - 1],
                recv_sem=ag_recv.at[off - 1],
                device_id=(peer,),
                device_id_type=pl.DeviceIdType.MESH,
            )
            rdma.start()
            rdmas.append(rdma)
        for rdma in rdmas:
            rdma.wait()

    hdn = jnp.maximum(
        jnp.dot(xfull_ref[:, :], win_ref[:, :],
                preferred_element_type=jnp.float32),
        0.0,
    )
    contrib = jnp.dot(hdn, wout_ref[:, :], preferred_element_type=jnp.float32)

    @pl.when(k == 0)
    def _init():
        acc_ref[:, :] = contrib

    @pl.when(k > 0)
    def _accum():
        acc_ref[:, :] = acc_ref[:, :] + contrib

    @pl.when(k == K_TILES - 1)
    def _rs():
        rdmas = []
        for off in range(1, N_DEV):
            peer = lax.rem(my + off, N_DEV)
            rdma = pltpu.make_async_remote_copy(
                src_ref=acc_ref.at[pl.ds(peer * m_per, m_per), :],
                dst_ref=comm_ref.at[off - 1],
                send_sem=rs_send.at[off - 1],
                recv_sem=rs_recv.at[off - 1],
                device_id=(peer,),
                device_id_type=pl.DeviceIdType.MESH,
            )
            rdma.start()
            rdmas.append(rdma)
        for rdma in rdmas:
            rdma.wait()

        out_ref[:, :] = (acc_ref[pl.ds(my * m_per, m_per), :]
                         + comm_ref[0] + comm_ref[1] + comm_ref[2])


def _layer(x_shard, win, wout, cid):
    m_per, d = x_shard.shape
    h_loc = win.shape[1]
    h_tile = h_loc // K_TILES
    return pl.pallas_call(
        _layer_body,
        grid=(K_TILES,),
        out_shape=jax.ShapeDtypeStruct((m_per, d), jnp.float32),
        in_specs=[
            pl.BlockSpec((m_per, d), lambda k: (0, 0)),
            pl.BlockSpec((d, h_tile), lambda k: (0, k)),
            pl.BlockSpec((h_tile, d), lambda k: (k, 0)),
        ],
        out_specs=pl.BlockSpec((m_per, d), lambda k: (0, 0)),
        scratch_shapes=[
            pltpu.VMEM((N_DEV * m_per, d), jnp.float32),
            pltpu.VMEM((N_DEV * m_per, d), jnp.float32),
            pltpu.VMEM((N_DEV - 1, m_per, d), jnp.float32),
            pltpu.SemaphoreType.DMA((N_DEV - 1,)),
            pltpu.SemaphoreType.DMA((N_DEV - 1,)),
            pltpu.SemaphoreType.DMA((N_DEV - 1,)),
            pltpu.SemaphoreType.DMA((N_DEV - 1,)),
        ],
        compiler_params=pltpu.CompilerParams(
            collective_id=cid,
            dimension_semantics=("arbitrary",),
        ),
    )(x_shard, win, wout)


def kernel(x, Win0, Wout0, Win1, Wout1, Win2, Wout2):
    x = _layer(x, Win0, Wout0, 0)
    x = _layer(x, Win1, Wout1, 1)
    x = _layer(x, Win2, Wout2, 2)
    return x


# device time: 144183 ns/iter; 1.4668x vs baseline; 1.1127x over previous
import jax
import jax.numpy as jnp
from jax import lax
from jax.experimental import pallas as pl
from jax.experimental.pallas import tpu as pltpu

N_DEV = 4
KW = 512
GW = 256


def _wait_recv(dst_slice, sem):
    pltpu.make_async_remote_copy(
        src_ref=dst_slice, dst_ref=dst_slice,
        send_sem=sem, recv_sem=sem,
        device_id=(0,), device_id_type=pl.DeviceIdType.MESH,
    ).wait_recv()


def _wait_send(src_slice, sem):
    pltpu.make_async_remote_copy(
        src_ref=src_slice, dst_ref=src_slice,
        send_sem=sem, recv_sem=sem,
        device_id=(0,), device_id_type=pl.DeviceIdType.MESH,
    ).wait_send()


def _make_body(first, last, m_per, d, h_loc):
    K1 = h_loc // KW
    G = d // GW

    def body(x_ref, win_ref, wout_ref, out_ref,
             xfull_ref, h_ref, pbuf_ref, rs_comm_ref, xnext_ref,
             ag_send, ag_recv, rs_send, rs_recv, bc_send, bc_recv):
        s = pl.program_id(0)
        my = lax.axis_index("i")

        @pl.when(s == 0)
        def _entry():
            barrier = pltpu.get_barrier_semaphore()
            for off in range(1, N_DEV):
                peer = lax.rem(my + off, N_DEV)
                pl.semaphore_signal(barrier, inc=1, device_id=(peer,),
                                    device_id_type=pl.DeviceIdType.MESH)
            pl.semaphore_wait(barrier, N_DEV - 1)

            if first:
                xfull_ref[pl.ds(my * m_per, m_per), :] = x_ref[:, :]
                rdmas = []
                for off in range(1, N_DEV):
                    peer = lax.rem(my + off, N_DEV)
                    rdma = pltpu.make_async_remote_copy(
                        src_ref=xfull_ref.at[pl.ds(my * m_per, m_per), :],
                        dst_ref=xfull_ref.at[pl.ds(my * m_per, m_per), :],
                        send_sem=ag_send.at[off - 1],
                        recv_sem=ag_recv.at[off - 1],
                        device_id=(peer,),
                        device_id_type=pl.DeviceIdType.MESH,
                    )
                    rdma.start()
                    rdmas.append(rdma)
                for rdma in rdmas:
                    rdma.wait()

        src_ref = xfull_ref if first else x_ref

        @pl.when(s < K1)
        def _pass1():
            h_ref[:, pl.ds(s * KW, KW)] = jnp.maximum(
                jnp.dot(src_ref[:, :], win_ref[:, :],
                        preferred_element_type=jnp.float32),
                0.0,
            )

        def combine(gc):
            for off in range(1, N_DEV):
                _wait_recv(rs_comm_ref.at[gc, off - 1], rs_recv.at[gc, off - 1])
            total = (pbuf_ref[gc, pl.ds(my * m_per, m_per), :]
                     + rs_comm_ref[gc, 0] + rs_comm_ref[gc, 1]
                     + rs_comm_ref[gc, 2])
            if last:
                out_ref[:, pl.ds(gc * GW, GW)] = total
            else:
                xnext_ref[pl.ds(my * m_per, m_per), pl.ds(gc * GW, GW)] = total
                for off in range(1, N_DEV):
                    peer = lax.rem(my + off, N_DEV)
                    pltpu.make_async_remote_copy(
                        src_ref=xnext_ref.at[pl.ds(my * m_per, m_per),
                                             pl.ds(gc * GW, GW)],
                        dst_ref=xnext_ref.at[pl.ds(my * m_per, m_per),
                                             pl.ds(gc * GW, GW)],
                        send_sem=bc_send.at[gc, off - 1],
                        recv_sem=bc_recv.at[gc, off - 1],
                        device_id=(peer,),
                        device_id_type=pl.DeviceIdType.MESH,
                    ).start()

        @pl.when(s >= K1)
        def _pass2():
            g = s - K1
            pbuf_ref[g, :, :] = jnp.dot(h_ref[:, :], wout_ref[:, :],
                                        preferred_element_type=jnp.float32)
            for off in range(1, N_DEV):
                peer = lax.rem(my + off, N_DEV)
                pltpu.make_async_remote_copy(
                    src_ref=pbuf_ref.at[g, pl.ds(peer * m_per, m_per), :],
                    dst_ref=rs_comm_ref.at[g, off - 1],
                    send_sem=rs_send.at[g, off - 1],
                    recv_sem=rs_recv.at[g, off - 1],
                    device_id=(peer,),
                    device_id_type=pl.DeviceIdType.MESH,
                ).start()

            @pl.when(g > 0)
            def _combine_prev():
                combine(g - 1)

        @pl.when(s == K1 + G - 1)
        def _tail():
            combine(G - 1)
            if not last:
                for g in range(G):
                    for off in range(1, N_DEV):
                        sender = lax.rem(my + N_DEV - off, N_DEV)
                        _wait_recv(
                            xnext_ref.at[pl.ds(sender * m_per, m_per),
                                         pl.ds(g * GW, GW)],
                            bc_recv.at[g, off - 1],
                        )
                out_ref[:, :] = xnext_ref[:, :]
            for g in range(G):
                for off in range(1, N_DEV):
                    _wait_send(pbuf_ref.at[g, pl.ds(0, m_per), :],
                               rs_send.at[g, off - 1])
                    if not last:
                        _wait_send(
                            xnext_ref.at[pl.ds(0, m_per), pl.ds(g * GW, GW)],
                            bc_send.at[g, off - 1],
                        )

    return body, K1, G


def _layer(x, win, wout, cid, first, last):
    d, h_loc = win.shape
    m_per = x.shape[0] if first else x.shape[0] // N_DEV
    m_full = N_DEV * m_per
    body, K1, G = _make_body(first, last, m_per, d, h_loc)

    x_block = (m_per, d) if first else (m_full, d)
    out_rows = m_per if last else m_full

    return pl.pallas_call(
        body,
        grid=(K1 + G,),
        out_shape=jax.ShapeDtypeStruct((out_rows, d), jnp.float32),
        in_specs=[
            pl.BlockSpec(x_block, lambda s: (0, 0)),
            pl.BlockSpec((d, KW), lambda s: (0, jnp.minimum(s, K1 - 1))),
            pl.BlockSpec((h_loc, GW), lambda s: (0, jnp.maximum(s - K1, 0))),
        ],
        out_specs=pl.BlockSpec((out_rows, d), lambda s: (0, 0)),
        scratch_shapes=[
            pltpu.VMEM((m_full, d), jnp.float32),
            pltpu.VMEM((m_full, h_loc), jnp.float32),
            pltpu.VMEM((G, m_full, GW), jnp.float32),
            pltpu.VMEM((G, N_DEV - 1, m_per, GW), jnp.float32),
            pltpu.VMEM((m_full, d), jnp.float32),
            pltpu.SemaphoreType.DMA((N_DEV - 1,)),
            pltpu.SemaphoreType.DMA((N_DEV - 1,)),
            pltpu.SemaphoreType.DMA((G, N_DEV - 1)),
            pltpu.SemaphoreType.DMA((G, N_DEV - 1)),
            pltpu.SemaphoreType.DMA((G, N_DEV - 1)),
            pltpu.SemaphoreType.DMA((G, N_DEV - 1)),
        ],
        compiler_params=pltpu.CompilerParams(
            collective_id=cid,
            dimension_semantics=("arbitrary",),
        ),
    )(x, win, wout)


def kernel(x, Win0, Wout0, Win1, Wout1, Win2, Wout2):
    x = _layer(x, Win0, Wout0, 0, first=True, last=False)
    x = _layer(x, Win1, Wout1, 1, first=False, last=False)
    x = _layer(x, Win2, Wout2, 2, first=False, last=True)
    return x
